# baseline (speedup 1.0000x reference)
import functools

import jax
import jax.numpy as jnp
from jax import lax
from jax.experimental import pallas as pl
from jax.experimental.pallas import tpu as pltpu

_K = 512
_D = 576
_COUT = 128
_TL = 112
_LT = 7


def _unfold_patches(x):
    B, C, H, W = x.shape
    k, pad, stride = 3, 1, 1
    xp = jnp.pad(x, ((0, 0), (0, 0), (pad, pad), (pad, pad)))
    outH = (H + 2 * pad - k) // stride + 1
    outW = (W + 2 * pad - k) // stride + 1
    cols = []
    for kh in range(k):
        for kw in range(k):
            cols.append(xp[:, :, kh:kh + outH * stride:stride, kw:kw + outW * stride:stride])
    p = jnp.stack(cols, axis=2)
    p = p.reshape(B, C * k * k, outH * outW)
    p = jnp.transpose(p, (0, 2, 1))
    return p, outH, outW


def _argmin_loop(q_ref, tab_ref, n_rows):
    tl = q_ref.shape[0]

    def body(k, carry):
        bd, bi = carry
        row = tab_ref[pl.ds(k, 1), :]
        d = jnp.sum(jnp.abs(q_ref[...] - row), axis=1)
        better = d < bd
        bi = jnp.where(better, k, bi)
        bd = jnp.where(better, d, bd)
        return bd, bi

    init = (jnp.full((tl,), 3e38, jnp.float32), jnp.zeros((tl,), jnp.int32))
    _, bi = lax.fori_loop(0, n_rows, body, init)
    return bi


def _vq_body(p_ref, c_ref, rc_ref, dc_ref, drc_ref,
             vals_ref, stats_ref, resid_ref, acc_ref):
    b = pl.program_id(0)
    lt = pl.program_id(1)
    tl = p_ref.shape[0]

    bi1 = _argmin_loop(p_ref, c_ref, _K)

    iota_k = lax.broadcasted_iota(jnp.int32, (tl, _K), 1)
    oh1 = (iota_k == bi1[:, None]).astype(jnp.float32)
    sel = lax.dot_general(oh1, c_ref[...], (((1,), (0,)), ((), ())),
                          precision=lax.Precision.HIGHEST,
                          preferred_element_type=jnp.float32)
    resid_ref[...] = p_ref[...] - sel

    bi2 = _argmin_loop(resid_ref, rc_ref, _K)
    oh2 = (iota_k == bi2[:, None]).astype(jnp.float32)

    vals = (lax.dot_general(oh1, dc_ref[...], (((1,), (0,)), ((), ())),
                            precision=lax.Precision.HIGHEST,
                            preferred_element_type=jnp.float32)
            + lax.dot_general(oh2, drc_ref[...], (((1,), (0,)), ((), ())),
                              precision=lax.Precision.HIGHEST,
                              preferred_element_type=jnp.float32))
    vals_ref[...] = vals

    part = jnp.concatenate(
        [jnp.sum(vals, axis=0)[None, :],
         jnp.sum(vals * vals, axis=0)[None, :],
         jnp.zeros((6, _COUT), jnp.float32)], axis=0)

    @pl.when(jnp.logical_and(b == 0, lt == 0))
    def _():
        acc_ref[...] = jnp.zeros_like(acc_ref)

    acc_ref[...] += part

    @pl.when(jnp.logical_and(b == pl.num_programs(0) - 1,
                             lt == pl.num_programs(1) - 1))
    def _():
        stats_ref[...] = acc_ref[...]


def _bn_body(vals_ref, stats_ref, g_ref, b_ref, out_ref):
    n = jnp.float32(vals_ref.shape[1] * pl.num_programs(0))
    s = stats_ref[0:1, :]
    ss = stats_ref[1:2, :]
    mean = s / n
    var = ss / n - mean * mean
    scale = g_ref[...] / jnp.sqrt(var + 1e-5)
    shift = b_ref[...] - mean * scale
    y = vals_ref[0] * scale + shift
    out_ref[0] = y.T


@jax.jit
def kernel(x, centroids, residual_centroids, dot_centroids,
           dot_residual_centroids, bn_gamma, bn_beta):
    B = x.shape[0]
    patches, outH, outW = _unfold_patches(x)
    L = patches.shape[1]
    patches = patches.reshape(B * L, _D)

    nbt = L // _TL
    vals, stats = pl.pallas_call(
        _vq_body,
        grid=(B, nbt),
        in_specs=[
            pl.BlockSpec((_TL, _D), lambda b, lt: (b * nbt + lt, 0)),
            pl.BlockSpec((_K, _D), lambda b, lt: (0, 0)),
            pl.BlockSpec((_K, _D), lambda b, lt: (0, 0)),
            pl.BlockSpec((_K, _COUT), lambda b, lt: (0, 0)),
            pl.BlockSpec((_K, _COUT), lambda b, lt: (0, 0)),
        ],
        out_specs=[
            pl.BlockSpec((_TL, _COUT), lambda b, lt: (b * nbt + lt, 0)),
            pl.BlockSpec((8, _COUT), lambda b, lt: (0, 0)),
        ],
        out_shape=[
            jax.ShapeDtypeStruct((B * L, _COUT), jnp.float32),
            jax.ShapeDtypeStruct((8, _COUT), jnp.float32),
        ],
        scratch_shapes=[
            pltpu.VMEM((_TL, _D), jnp.float32),
            pltpu.VMEM((8, _COUT), jnp.float32),
        ],
    )(patches, centroids, residual_centroids, dot_centroids,
      dot_residual_centroids)

    out = pl.pallas_call(
        _bn_body,
        grid=(B,),
        in_specs=[
            pl.BlockSpec((1, L, _COUT), lambda b: (b, 0, 0)),
            pl.BlockSpec((8, _COUT), lambda b: (0, 0)),
            pl.BlockSpec((1, _COUT), lambda b: (0, 0)),
            pl.BlockSpec((1, _COUT), lambda b: (0, 0)),
        ],
        out_specs=pl.BlockSpec((1, _COUT, L), lambda b: (b, 0, 0)),
        out_shape=jax.ShapeDtypeStruct((B, _COUT, L), jnp.float32),
    )(vals.reshape(B, L, _COUT), stats,
      bn_gamma.reshape(1, _COUT), bn_beta.reshape(1, _COUT))

    return out.reshape(B, _COUT, outH, outW)


# speedup vs baseline: 2.1001x; 2.1001x over previous
import jax
import jax.numpy as jnp
from jax import lax
from jax.experimental import pallas as pl
from jax.experimental.pallas import tpu as pltpu

_K = 512
_D = 576
_DB = _D // 8
_COUT = 128
_TL = 64
_NT = 49
_BIGF = 3e38
_BIGI = 1 << 30


def _unfold_patches(x):
    B, C, H, W = x.shape
    k, pad, stride = 3, 1, 1
    xp = jnp.pad(x, ((0, 0), (0, 0), (pad, pad), (pad, pad)))
    outH = (H + 2 * pad - k) // stride + 1
    outW = (W + 2 * pad - k) // stride + 1
    cols = []
    for kh in range(k):
        for kw in range(k):
            cols.append(xp[:, :, kh:kh + outH * stride:stride, kw:kw + outW * stride:stride])
    p = jnp.stack(cols, axis=2)
    p = p.reshape(B, C * k * k, outH * outW)
    p = jnp.transpose(p, (0, 2, 1))
    return p, outH, outW


def _dot_hi(a, b_ref):
    return lax.dot_general(a, b_ref[...], (((1,), (0,)), ((), ())),
                           precision=lax.Precision.HIGHEST,
                           preferred_element_type=jnp.float32)


def _vq_body(p_ref, cT_ref, c_ref, rcT_ref, rc_ref, dc_ref, drc_ref,
             vals_ref, stats_ref, p3_ref, resid_ref, acc_ref):
    i = pl.program_id(0)
    iota = lax.broadcasted_iota(jnp.int32, (_TL, _K), 1)

    def dist_matrix(tabT_ref):
        def step(db, acc):
            p8 = p3_ref[db]
            c8 = tabT_ref[pl.ds(db * 8, 8), :]
            for j in range(8):
                acc = acc + jnp.abs(p8[:, j:j + 1] - c8[j:j + 1, :])
            return acc
        return lax.fori_loop(0, _DB, step, jnp.zeros((_TL, _K), jnp.float32))

    def top2(acc):
        m1 = jnp.min(acc, axis=1)
        i1 = jnp.min(jnp.where(acc == m1[:, None], iota, _BIGI), axis=1)
        accx = jnp.where(iota == i1[:, None], _BIGF, acc)
        m2 = jnp.min(accx, axis=1)
        i2 = jnp.min(jnp.where(accx == m2[:, None], iota, _BIGI), axis=1)
        return i1, i2

    def refine(q, tab_ref, ia, ib):
        ra = _dot_hi((iota == ia[:, None]).astype(jnp.float32), tab_ref)
        rb = _dot_hi((iota == ib[:, None]).astype(jnp.float32), tab_ref)
        da = jnp.sum(jnp.abs(q - ra), axis=1)
        db_ = jnp.sum(jnp.abs(q - rb), axis=1)
        swap = (db_ < da) | ((db_ == da) & (ib < ia))
        fi = jnp.where(swap, ib, ia)
        row = jnp.where(swap[:, None], rb, ra)
        return fi, row

    for db in range(_DB):
        p3_ref[db] = p_ref[:, db * 8:(db + 1) * 8]

    acc1 = dist_matrix(cT_ref)
    i1a, i1b = top2(acc1)
    p = p_ref[...]
    fi1, sel = refine(p, c_ref, i1a, i1b)
    resid_ref[...] = p - sel

    for db in range(_DB):
        p3_ref[db] = resid_ref[:, db * 8:(db + 1) * 8]

    acc2 = dist_matrix(rcT_ref)
    i2a, i2b = top2(acc2)
    fi2, _ = refine(resid_ref[...], rc_ref, i2a, i2b)

    oh1 = (iota == fi1[:, None]).astype(jnp.float32)
    oh2 = (iota == fi2[:, None]).astype(jnp.float32)
    vals = _dot_hi(oh1, dc_ref) + _dot_hi(oh2, drc_ref)
    vals_ref[...] = vals

    part = jnp.concatenate(
        [jnp.sum(vals, axis=0)[None, :],
         jnp.sum(vals * vals, axis=0)[None, :],
         jnp.zeros((6, _COUT), jnp.float32)], axis=0)

    @pl.when(i == 0)
    def _():
        acc_ref[...] = jnp.zeros_like(acc_ref)

    acc_ref[...] += part

    @pl.when(i == _NT - 1)
    def _():
        stats_ref[...] = acc_ref[...]


def _bn_body(vals_ref, stats_ref, g_ref, b_ref, out_ref):
    n = jnp.float32(vals_ref.shape[1] * pl.num_programs(0))
    s = stats_ref[0:1, :]
    ss = stats_ref[1:2, :]
    mean = s / n
    var = ss / n - mean * mean
    scale = g_ref[...] / jnp.sqrt(var + 1e-5)
    shift = b_ref[...] - mean * scale
    y = vals_ref[0] * scale + shift
    out_ref[0] = y.T


def kernel(x, centroids, residual_centroids, dot_centroids,
           dot_residual_centroids, bn_gamma, bn_beta):
    B = x.shape[0]
    patches, outH, outW = _unfold_patches(x)
    L = patches.shape[1]
    patches = patches.reshape(B * L, _D)
    cT = centroids.T
    rcT = residual_centroids.T

    vals, stats = pl.pallas_call(
        _vq_body,
        grid=(_NT,),
        in_specs=[
            pl.BlockSpec((_TL, _D), lambda i: (i, 0)),
            pl.BlockSpec((_D, _K), lambda i: (0, 0)),
            pl.BlockSpec((_K, _D), lambda i: (0, 0)),
            pl.BlockSpec((_D, _K), lambda i: (0, 0)),
            pl.BlockSpec((_K, _D), lambda i: (0, 0)),
            pl.BlockSpec((_K, _COUT), lambda i: (0, 0)),
            pl.BlockSpec((_K, _COUT), lambda i: (0, 0)),
        ],
        out_specs=[
            pl.BlockSpec((_TL, _COUT), lambda i: (i, 0)),
            pl.BlockSpec((8, _COUT), lambda i: (0, 0)),
        ],
        out_shape=[
            jax.ShapeDtypeStruct((B * L, _COUT), jnp.float32),
            jax.ShapeDtypeStruct((8, _COUT), jnp.float32),
        ],
        scratch_shapes=[
            pltpu.VMEM((_DB, _TL, 8), jnp.float32),
            pltpu.VMEM((_TL, _D), jnp.float32),
            pltpu.VMEM((8, _COUT), jnp.float32),
        ],
    )(patches, cT, centroids, rcT, residual_centroids, dot_centroids,
      dot_residual_centroids)

    out = pl.pallas_call(
        _bn_body,
        grid=(B,),
        in_specs=[
            pl.BlockSpec((1, L, _COUT), lambda b: (b, 0, 0)),
            pl.BlockSpec((8, _COUT), lambda b: (0, 0)),
            pl.BlockSpec((1, _COUT), lambda b: (0, 0)),
            pl.BlockSpec((1, _COUT), lambda b: (0, 0)),
        ],
        out_specs=pl.BlockSpec((1, _COUT, L), lambda b: (b, 0, 0)),
        out_shape=jax.ShapeDtypeStruct((B, _COUT, L), jnp.float32),
    )(vals.reshape(B, L, _COUT), stats,
      bn_gamma.reshape(1, _COUT), bn_beta.reshape(1, _COUT))

    return out.reshape(B, _COUT, outH, outW)


# speedup vs baseline: 2.5110x; 1.1956x over previous
import jax
import jax.numpy as jnp
from jax import lax
from jax.experimental import pallas as pl
from jax.experimental.pallas import tpu as pltpu

_K = 512
_D = 576
_DB = _D // 8
_COUT = 128
_TL = 64
_NT = 49
_BIGF = 3e38
_BIGI = 1 << 30


def _unfold_patches(x):
    B, C, H, W = x.shape
    k, pad, stride = 3, 1, 1
    xp = jnp.pad(x, ((0, 0), (0, 0), (pad, pad), (pad, pad)))
    outH = (H + 2 * pad - k) // stride + 1
    outW = (W + 2 * pad - k) // stride + 1
    cols = []
    for kh in range(k):
        for kw in range(k):
            cols.append(xp[:, :, kh:kh + outH * stride:stride, kw:kw + outW * stride:stride])
    p = jnp.stack(cols, axis=2)
    p = p.reshape(B, C * k * k, outH * outW)
    p = jnp.transpose(p, (0, 2, 1))
    return p, outH, outW


def _split3(t):
    bits = lax.bitcast_convert_type(t, jnp.uint32)
    hi = lax.bitcast_convert_type(bits & jnp.uint32(0xFFFF0000), jnp.float32)
    r1 = t - hi
    b1 = lax.bitcast_convert_type(r1, jnp.uint32)
    mid = lax.bitcast_convert_type(b1 & jnp.uint32(0xFFFF0000), jnp.float32)
    lo = r1 - mid
    return jnp.stack([hi.astype(jnp.bfloat16), mid.astype(jnp.bfloat16),
                      lo.astype(jnp.bfloat16)])


def _dot3(ohb, t3_ref):
    def d(piece):
        return lax.dot_general(ohb, piece, (((1,), (0,)), ((), ())),
                               preferred_element_type=jnp.float32)
    return (d(t3_ref[0]) + d(t3_ref[1])) + d(t3_ref[2])


def _vq_body(p_ref, c3b_ref, rc3b_ref, c3_ref, rc3_ref, dc3_ref, drc3_ref,
             vals_ref, stats_ref, p3_ref, resid_ref, acc_ref):
    i = pl.program_id(0)
    iota = lax.broadcasted_iota(jnp.int32, (_TL, _K), 1)

    def dist_matrix(tab3_ref):
        def step(s, acc):
            for t in range(2):
                db = s * 2 + t
                p8 = p3_ref[db]
                c8 = tab3_ref[db]
                for j in range(8):
                    acc = acc + jnp.abs(p8[:, j:j + 1] - c8[j:j + 1, :])
            return acc
        return lax.fori_loop(0, _DB // 2, step,
                             jnp.zeros((_TL, _K), jnp.float32))

    def top2(acc):
        m1 = jnp.min(acc, axis=1)
        i1 = jnp.min(jnp.where(acc == m1[:, None], iota, _BIGI), axis=1)
        accx = jnp.where(iota == i1[:, None], _BIGF, acc)
        m2 = jnp.min(accx, axis=1)
        i2 = jnp.min(jnp.where(accx == m2[:, None], iota, _BIGI), axis=1)
        return i1, i2

    def refine(q, t3_ref, ia, ib):
        ra = _dot3((iota == ia[:, None]).astype(jnp.bfloat16), t3_ref)
        rb = _dot3((iota == ib[:, None]).astype(jnp.bfloat16), t3_ref)
        da = jnp.sum(jnp.abs(q - ra), axis=1)
        db_ = jnp.sum(jnp.abs(q - rb), axis=1)
        swap = (db_ < da) | ((db_ == da) & (ib < ia))
        fi = jnp.where(swap, ib, ia)
        row = jnp.where(swap[:, None], rb, ra)
        return fi, row

    for db in range(_DB):
        p3_ref[db] = p_ref[:, db * 8:(db + 1) * 8]

    acc1 = dist_matrix(c3b_ref)
    i1a, i1b = top2(acc1)
    p = p_ref[...]
    fi1, sel = refine(p, c3_ref, i1a, i1b)
    resid = p - sel
    resid_ref[...] = resid

    for db in range(_DB):
        p3_ref[db] = resid[:, db * 8:(db + 1) * 8]

    acc2 = dist_matrix(rc3b_ref)
    i2a, i2b = top2(acc2)
    fi2, _ = refine(resid_ref[...], rc3_ref, i2a, i2b)

    oh1 = (iota == fi1[:, None]).astype(jnp.bfloat16)
    oh2 = (iota == fi2[:, None]).astype(jnp.bfloat16)
    vals = _dot3(oh1, dc3_ref) + _dot3(oh2, drc3_ref)
    vals_ref[...] = vals

    part = jnp.concatenate(
        [jnp.sum(vals, axis=0)[None, :],
         jnp.sum(vals * vals, axis=0)[None, :],
         jnp.zeros((6, _COUT), jnp.float32)], axis=0)

    @pl.when(i == 0)
    def _():
        acc_ref[...] = jnp.zeros_like(acc_ref)

    acc_ref[...] += part

    @pl.when(i == _NT - 1)
    def _():
        stats_ref[...] = acc_ref[...]


def _bn_body(vals_ref, stats_ref, g_ref, b_ref, out_ref):
    n = jnp.float32(vals_ref.shape[1] * pl.num_programs(0))
    s = stats_ref[0:1, :]
    ss = stats_ref[1:2, :]
    mean = s / n
    var = ss / n - mean * mean
    scale = g_ref[...] / jnp.sqrt(var + 1e-5)
    shift = b_ref[...] - mean * scale
    y = vals_ref[0] * scale + shift
    out_ref[0] = y.T


def kernel(x, centroids, residual_centroids, dot_centroids,
           dot_residual_centroids, bn_gamma, bn_beta):
    B = x.shape[0]
    patches, outH, outW = _unfold_patches(x)
    L = patches.shape[1]
    patches = patches.reshape(B * L, _D)
    c3b = centroids.T.reshape(_DB, 8, _K)
    rc3b = residual_centroids.T.reshape(_DB, 8, _K)
    c3 = _split3(centroids)
    rc3 = _split3(residual_centroids)
    dc3 = _split3(dot_centroids)
    drc3 = _split3(dot_residual_centroids)

    full = lambda shape: pl.BlockSpec(shape, lambda i: tuple(0 for _ in shape))
    vals, stats = pl.pallas_call(
        _vq_body,
        grid=(_NT,),
        in_specs=[
            pl.BlockSpec((_TL, _D), lambda i: (i, 0)),
            full((_DB, 8, _K)),
            full((_DB, 8, _K)),
            full((3, _K, _D)),
            full((3, _K, _D)),
            full((3, _K, _COUT)),
            full((3, _K, _COUT)),
        ],
        out_specs=[
            pl.BlockSpec((_TL, _COUT), lambda i: (i, 0)),
            pl.BlockSpec((8, _COUT), lambda i: (0, 0)),
        ],
        out_shape=[
            jax.ShapeDtypeStruct((B * L, _COUT), jnp.float32),
            jax.ShapeDtypeStruct((8, _COUT), jnp.float32),
        ],
        scratch_shapes=[
            pltpu.VMEM((_DB, _TL, 8), jnp.float32),
            pltpu.VMEM((_TL, _D), jnp.float32),
            pltpu.VMEM((8, _COUT), jnp.float32),
        ],
    )(patches, c3b, rc3b, c3, rc3, dc3, drc3)

    out = pl.pallas_call(
        _bn_body,
        grid=(B,),
        in_specs=[
            pl.BlockSpec((1, L, _COUT), lambda b: (b, 0, 0)),
            pl.BlockSpec((8, _COUT), lambda b: (0, 0)),
            pl.BlockSpec((1, _COUT), lambda b: (0, 0)),
            pl.BlockSpec((1, _COUT), lambda b: (0, 0)),
        ],
        out_specs=pl.BlockSpec((1, _COUT, L), lambda b: (b, 0, 0)),
        out_shape=jax.ShapeDtypeStruct((B, _COUT, L), jnp.float32),
    )(vals.reshape(B, L, _COUT), stats,
      bn_gamma.reshape(1, _COUT), bn_beta.reshape(1, _COUT))

    return out.reshape(B, _COUT, outH, outW)


# speedup vs baseline: 2.8690x; 1.1426x over previous
import jax
import jax.numpy as jnp
from jax import lax
from jax.experimental import pallas as pl
from jax.experimental.pallas import tpu as pltpu

_K = 512
_D = 576
_DB = _D // 8
_COUT = 128
_TL = 64
_NT = 49
_BIGF = 3e38


def _unfold_patches_dl(x):
    B, C, H, W = x.shape
    k, pad, stride = 3, 1, 1
    xp = jnp.pad(x, ((0, 0), (0, 0), (pad, pad), (pad, pad)))
    outH = (H + 2 * pad - k) // stride + 1
    outW = (W + 2 * pad - k) // stride + 1
    cols = []
    for kh in range(k):
        for kw in range(k):
            cols.append(xp[:, :, kh:kh + outH * stride:stride, kw:kw + outW * stride:stride])
    p = jnp.stack(cols, axis=2)
    p = p.reshape(B, C * k * k, outH * outW)
    return p, outH, outW


def _split3(t):
    bits = lax.bitcast_convert_type(t, jnp.uint32)
    hi = lax.bitcast_convert_type(bits & jnp.uint32(0xFFFF0000), jnp.float32)
    r1 = t - hi
    b1 = lax.bitcast_convert_type(r1, jnp.uint32)
    mid = lax.bitcast_convert_type(b1 & jnp.uint32(0xFFFF0000), jnp.float32)
    lo = r1 - mid
    return jnp.stack([hi.astype(jnp.bfloat16), mid.astype(jnp.bfloat16),
                      lo.astype(jnp.bfloat16)])


def _dot3(ohb, t3_ref):
    def d(piece):
        return lax.dot_general(ohb, piece, (((1,), (0,)), ((), ())),
                               preferred_element_type=jnp.float32)
    return (d(t3_ref[0]) + d(t3_ref[1])) + d(t3_ref[2])


def _prep_body(pdl_ref, c_ref, rc_ref, pt_ref, ct_ref, rct_ref):
    b = pl.program_id(0)
    pt_ref[...] = pdl_ref[...].T

    @pl.when(b == 0)
    def _():
        ct_ref[...] = c_ref[...].T
        rct_ref[...] = rc_ref[...].T


def _vq_body(p_ref, c3b_ref, rc3b_ref, c3_ref, rc3_ref, dc3_ref, drc3_ref,
             vals_ref, stats_ref, p3_ref, resid_ref, acc_ref):
    i = pl.program_id(0)
    iota = lax.broadcasted_iota(jnp.int32, (_TL, _K), 1).astype(jnp.float32)

    def dist_matrix(tab3_ref):
        def step(s, acc):
            for t in range(4):
                db = s * 4 + t
                p8 = p3_ref[db]
                c8 = tab3_ref[db]
                for j in range(8):
                    acc = acc + jnp.abs(p8[:, j:j + 1] - c8[j:j + 1, :])
            return acc
        return lax.fori_loop(0, _DB // 4, step,
                             jnp.zeros((_TL, _K), jnp.float32))

    def top2(acc):
        m1 = jnp.min(acc, axis=1)
        i1 = jnp.min(jnp.where(acc == m1[:, None], iota, _BIGF), axis=1)
        accx = jnp.where(iota == i1[:, None], _BIGF, acc)
        m2 = jnp.min(accx, axis=1)
        i2 = jnp.min(jnp.where(accx == m2[:, None], iota, _BIGF), axis=1)
        return i1, i2

    def refine(q, t3_ref, ia, ib):
        ra = _dot3((iota == ia[:, None]).astype(jnp.bfloat16), t3_ref)
        rb = _dot3((iota == ib[:, None]).astype(jnp.bfloat16), t3_ref)
        da = jnp.sum(jnp.abs(q - ra), axis=1)
        db_ = jnp.sum(jnp.abs(q - rb), axis=1)
        swap = (db_ < da) | ((db_ == da) & (ib < ia))
        fi = jnp.where(swap, ib, ia)
        row = jnp.where(swap[:, None], rb, ra)
        return fi, row

    for db in range(_DB):
        p3_ref[db] = p_ref[:, db * 8:(db + 1) * 8]

    acc1 = dist_matrix(c3b_ref)
    i1a, i1b = top2(acc1)
    p = p_ref[...]
    fi1, sel = refine(p, c3_ref, i1a, i1b)
    resid = p - sel
    resid_ref[...] = resid

    for db in range(_DB):
        p3_ref[db] = resid[:, db * 8:(db + 1) * 8]

    acc2 = dist_matrix(rc3b_ref)
    i2a, i2b = top2(acc2)
    fi2, _ = refine(resid_ref[...], rc3_ref, i2a, i2b)

    oh1 = (iota == fi1[:, None]).astype(jnp.bfloat16)
    oh2 = (iota == fi2[:, None]).astype(jnp.bfloat16)
    vals = _dot3(oh1, dc3_ref) + _dot3(oh2, drc3_ref)
    vals_ref[...] = vals

    part = jnp.concatenate(
        [jnp.sum(vals, axis=0)[None, :],
         jnp.sum(vals * vals, axis=0)[None, :],
         jnp.zeros((6, _COUT), jnp.float32)], axis=0)

    @pl.when(i == 0)
    def _():
        acc_ref[...] = jnp.zeros_like(acc_ref)

    acc_ref[...] += part

    @pl.when(i == _NT - 1)
    def _():
        stats_ref[...] = acc_ref[...]


def _bn_body(vals_ref, stats_ref, g_ref, b_ref, out_ref):
    n = jnp.float32(vals_ref.shape[1] * pl.num_programs(0))
    s = stats_ref[0:1, :]
    ss = stats_ref[1:2, :]
    mean = s / n
    var = ss / n - mean * mean
    scale = g_ref[...] / jnp.sqrt(var + 1e-5)
    shift = b_ref[...] - mean * scale
    y = vals_ref[0] * scale + shift
    out_ref[0] = y.T


def kernel(x, centroids, residual_centroids, dot_centroids,
           dot_residual_centroids, bn_gamma, bn_beta):
    B = x.shape[0]
    patches_dl, outH, outW = _unfold_patches_dl(x)
    L = patches_dl.shape[2]

    patches, cT, rcT = pl.pallas_call(
        _prep_body,
        grid=(B,),
        in_specs=[
            pl.BlockSpec((_D, L), lambda b: (b, 0)),
            pl.BlockSpec((_K, _D), lambda b: (0, 0)),
            pl.BlockSpec((_K, _D), lambda b: (0, 0)),
        ],
        out_specs=[
            pl.BlockSpec((L, _D), lambda b: (b, 0)),
            pl.BlockSpec((_D, _K), lambda b: (0, 0)),
            pl.BlockSpec((_D, _K), lambda b: (0, 0)),
        ],
        out_shape=[
            jax.ShapeDtypeStruct((B * L, _D), jnp.float32),
            jax.ShapeDtypeStruct((_D, _K), jnp.float32),
            jax.ShapeDtypeStruct((_D, _K), jnp.float32),
        ],
    )(patches_dl.reshape(B * _D, L), centroids, residual_centroids)

    c3b = cT.reshape(_DB, 8, _K)
    rc3b = rcT.reshape(_DB, 8, _K)
    c3 = _split3(centroids)
    rc3 = _split3(residual_centroids)
    dc3 = _split3(dot_centroids)
    drc3 = _split3(dot_residual_centroids)

    full = lambda shape: pl.BlockSpec(shape, lambda i: tuple(0 for _ in shape))
    vals, stats = pl.pallas_call(
        _vq_body,
        grid=(_NT,),
        in_specs=[
            pl.BlockSpec((_TL, _D), lambda i: (i, 0)),
            full((_DB, 8, _K)),
            full((_DB, 8, _K)),
            full((3, _K, _D)),
            full((3, _K, _D)),
            full((3, _K, _COUT)),
            full((3, _K, _COUT)),
        ],
        out_specs=[
            pl.BlockSpec((_TL, _COUT), lambda i: (i, 0)),
            pl.BlockSpec((8, _COUT), lambda i: (0, 0)),
        ],
        out_shape=[
            jax.ShapeDtypeStruct((B * L, _COUT), jnp.float32),
            jax.ShapeDtypeStruct((8, _COUT), jnp.float32),
        ],
        scratch_shapes=[
            pltpu.VMEM((_DB, _TL, 8), jnp.float32),
            pltpu.VMEM((_TL, _D), jnp.float32),
            pltpu.VMEM((8, _COUT), jnp.float32),
        ],
    )(patches, c3b, rc3b, c3, rc3, dc3, drc3)

    out = pl.pallas_call(
        _bn_body,
        grid=(B,),
        in_specs=[
            pl.BlockSpec((1, L, _COUT), lambda b: (b, 0, 0)),
            pl.BlockSpec((8, _COUT), lambda b: (0, 0)),
            pl.BlockSpec((1, _COUT), lambda b: (0, 0)),
            pl.BlockSpec((1, _COUT), lambda b: (0, 0)),
        ],
        out_specs=pl.BlockSpec((1, _COUT, L), lambda b: (b, 0, 0)),
        out_shape=jax.ShapeDtypeStruct((B, _COUT, L), jnp.float32),
    )(vals.reshape(B, L, _COUT), stats,
      bn_gamma.reshape(1, _COUT), bn_beta.reshape(1, _COUT))

    return out.reshape(B, _COUT, outH, outW)


# speedup vs baseline: 2.9203x; 1.0179x over previous
import jax
import jax.numpy as jnp
from jax import lax
from jax.experimental import pallas as pl
from jax.experimental.pallas import tpu as pltpu

_K = 512
_D = 576
_DB = _D // 8
_DB2 = _D // 16
_COUT = 128
_TL = 64
_NT = 49
_BIGF = 3e38


def _unfold_patches_dl(x):
    B, C, H, W = x.shape
    k, pad, stride = 3, 1, 1
    xp = jnp.pad(x, ((0, 0), (0, 0), (pad, pad), (pad, pad)))
    outH = (H + 2 * pad - k) // stride + 1
    outW = (W + 2 * pad - k) // stride + 1
    cols = []
    for kh in range(k):
        for kw in range(k):
            cols.append(xp[:, :, kh:kh + outH * stride:stride, kw:kw + outW * stride:stride])
    p = jnp.stack(cols, axis=2)
    p = p.reshape(B, C * k * k, outH * outW)
    return p, outH, outW


def _split3(t):
    bits = lax.bitcast_convert_type(t, jnp.uint32)
    hi = lax.bitcast_convert_type(bits & jnp.uint32(0xFFFF0000), jnp.float32)
    r1 = t - hi
    b1 = lax.bitcast_convert_type(r1, jnp.uint32)
    mid = lax.bitcast_convert_type(b1 & jnp.uint32(0xFFFF0000), jnp.float32)
    lo = r1 - mid
    return jnp.stack([hi.astype(jnp.bfloat16), mid.astype(jnp.bfloat16),
                      lo.astype(jnp.bfloat16)])


def _dot3(ohb, t3_ref):
    def d(piece):
        return lax.dot_general(ohb, piece, (((1,), (0,)), ((), ())),
                               preferred_element_type=jnp.float32)
    return (d(t3_ref[0]) + d(t3_ref[1])) + d(t3_ref[2])


def _prep_body(pdl_ref, c_ref, rc_ref, pt_ref, ct_ref, rct_ref):
    b = pl.program_id(0)
    pt_ref[...] = pdl_ref[...].T

    @pl.when(b == 0)
    def _():
        ct_ref[...] = c_ref[...].T
        rct_ref[...] = rc_ref[...].T


def _vq_body(p_ref, c3b_ref, rc3b_ref, c3_ref, rc3_ref, dc3_ref, drc3_ref,
             vals_ref, stats_ref, p3_ref, resid_ref, acc_ref):
    i = pl.program_id(0)
    iota = lax.broadcasted_iota(jnp.int32, (_TL, _K), 1).astype(jnp.float32)

    def dist_matrix(tab3_ref):
        def step(s, acc):
            for t in range(2):
                db = s * 2 + t
                p16 = p3_ref[db]
                c16 = tab3_ref[db]
                for j in range(16):
                    acc = acc + jnp.abs(p16[:, j:j + 1] - c16[j:j + 1, :])
            return acc
        return lax.fori_loop(0, _DB2 // 2, step,
                             jnp.zeros((_TL, _K), jnp.float32))

    def top2(acc):
        m1 = jnp.min(acc, axis=1)
        eq1 = acc == m1[:, None]
        i1 = jnp.min(jnp.where(eq1, iota, _BIGF), axis=1)
        accx = jnp.where(eq1, _BIGF, acc)
        m2 = jnp.min(accx, axis=1)
        i2 = jnp.min(jnp.where(accx == m2[:, None], iota, _BIGF), axis=1)
        return i1, i2

    def refine(q, t3_ref, ia, ib):
        ra = _dot3((iota == ia[:, None]).astype(jnp.bfloat16), t3_ref)
        rb = _dot3((iota == ib[:, None]).astype(jnp.bfloat16), t3_ref)
        da = jnp.sum(jnp.abs(q - ra), axis=1)
        db_ = jnp.sum(jnp.abs(q - rb), axis=1)
        swap = (db_ < da) | ((db_ == da) & (ib < ia))
        fi = jnp.where(swap, ib, ia)
        row = jnp.where(swap[:, None], rb, ra)
        return fi, row

    for db in range(_DB2):
        p3_ref[db] = p_ref[:, db * 16:(db + 1) * 16]

    acc1 = dist_matrix(c3b_ref)
    i1a, i1b = top2(acc1)
    p = p_ref[...]
    fi1, sel = refine(p, c3_ref, i1a, i1b)
    resid = p - sel
    resid_ref[...] = resid

    for db in range(_DB2):
        p3_ref[db] = resid[:, db * 16:(db + 1) * 16]

    acc2 = dist_matrix(rc3b_ref)
    i2a, i2b = top2(acc2)
    fi2, _ = refine(resid_ref[...], rc3_ref, i2a, i2b)

    oh1 = (iota == fi1[:, None]).astype(jnp.bfloat16)
    oh2 = (iota == fi2[:, None]).astype(jnp.bfloat16)
    vals = _dot3(oh1, dc3_ref) + _dot3(oh2, drc3_ref)
    vals_ref[...] = vals

    part = jnp.concatenate(
        [jnp.sum(vals, axis=0)[None, :],
         jnp.sum(vals * vals, axis=0)[None, :],
         jnp.zeros((6, _COUT), jnp.float32)], axis=0)

    @pl.when(i == 0)
    def _():
        acc_ref[...] = jnp.zeros_like(acc_ref)

    acc_ref[...] += part

    @pl.when(i == _NT - 1)
    def _():
        stats_ref[...] = acc_ref[...]


def _bn_body(vals_ref, stats_ref, g_ref, b_ref, out_ref):
    n = jnp.float32(vals_ref.shape[1] * pl.num_programs(0))
    s = stats_ref[0:1, :]
    ss = stats_ref[1:2, :]
    mean = s / n
    var = ss / n - mean * mean
    scale = g_ref[...] / jnp.sqrt(var + 1e-5)
    shift = b_ref[...] - mean * scale
    y = vals_ref[0] * scale + shift
    out_ref[0] = y.T


def kernel(x, centroids, residual_centroids, dot_centroids,
           dot_residual_centroids, bn_gamma, bn_beta):
    B = x.shape[0]
    patches_dl, outH, outW = _unfold_patches_dl(x)
    L = patches_dl.shape[2]

    patches, cT, rcT = pl.pallas_call(
        _prep_body,
        grid=(B,),
        in_specs=[
            pl.BlockSpec((_D, L), lambda b: (b, 0)),
            pl.BlockSpec((_K, _D), lambda b: (0, 0)),
            pl.BlockSpec((_K, _D), lambda b: (0, 0)),
        ],
        out_specs=[
            pl.BlockSpec((L, _D), lambda b: (b, 0)),
            pl.BlockSpec((_D, _K), lambda b: (0, 0)),
            pl.BlockSpec((_D, _K), lambda b: (0, 0)),
        ],
        out_shape=[
            jax.ShapeDtypeStruct((B * L, _D), jnp.float32),
            jax.ShapeDtypeStruct((_D, _K), jnp.float32),
            jax.ShapeDtypeStruct((_D, _K), jnp.float32),
        ],
    )(patches_dl.reshape(B * _D, L), centroids, residual_centroids)

    c3b = cT.reshape(_DB2, 16, _K)
    rc3b = rcT.reshape(_DB2, 16, _K)
    c3 = _split3(centroids)
    rc3 = _split3(residual_centroids)
    dc3 = _split3(dot_centroids)
    drc3 = _split3(dot_residual_centroids)

    full = lambda shape: pl.BlockSpec(shape, lambda i: tuple(0 for _ in shape))
    vals, stats = pl.pallas_call(
        _vq_body,
        grid=(_NT,),
        in_specs=[
            pl.BlockSpec((_TL, _D), lambda i: (i, 0)),
            full((_DB2, 16, _K)),
            full((_DB2, 16, _K)),
            full((3, _K, _D)),
            full((3, _K, _D)),
            full((3, _K, _COUT)),
            full((3, _K, _COUT)),
        ],
        out_specs=[
            pl.BlockSpec((_TL, _COUT), lambda i: (i, 0)),
            pl.BlockSpec((8, _COUT), lambda i: (0, 0)),
        ],
        out_shape=[
            jax.ShapeDtypeStruct((B * L, _COUT), jnp.float32),
            jax.ShapeDtypeStruct((8, _COUT), jnp.float32),
        ],
        scratch_shapes=[
            pltpu.VMEM((_DB2, _TL, 16), jnp.float32),
            pltpu.VMEM((_TL, _D), jnp.float32),
            pltpu.VMEM((8, _COUT), jnp.float32),
        ],
    )(patches, c3b, rc3b, c3, rc3, dc3, drc3)

    out = pl.pallas_call(
        _bn_body,
        grid=(B,),
        in_specs=[
            pl.BlockSpec((1, L, _COUT), lambda b: (b, 0, 0)),
            pl.BlockSpec((8, _COUT), lambda b: (0, 0)),
            pl.BlockSpec((1, _COUT), lambda b: (0, 0)),
            pl.BlockSpec((1, _COUT), lambda b: (0, 0)),
        ],
        out_specs=pl.BlockSpec((1, _COUT, L), lambda b: (b, 0, 0)),
        out_shape=jax.ShapeDtypeStruct((B, _COUT, L), jnp.float32),
    )(vals.reshape(B, L, _COUT), stats,
      bn_gamma.reshape(1, _COUT), bn_beta.reshape(1, _COUT))

    return out.reshape(B, _COUT, outH, outW)


# speedup vs baseline: 3.0895x; 1.0579x over previous
import jax
import jax.numpy as jnp
from jax import lax
from jax.experimental import pallas as pl
from jax.experimental.pallas import tpu as pltpu

_K = 512
_D = 576
_DB = _D // 8
_DB2 = _D // 16
_COUT = 128
_TL = 64
_NT = 49
_BIGF = 3e38


def _unfold_patches_dl(x):
    B, C, H, W = x.shape
    k, pad, stride = 3, 1, 1
    xp = jnp.pad(x, ((0, 0), (0, 0), (pad, pad), (pad, pad)))
    outH = (H + 2 * pad - k) // stride + 1
    outW = (W + 2 * pad - k) // stride + 1
    cols = []
    for kh in range(k):
        for kw in range(k):
            cols.append(xp[:, :, kh:kh + outH * stride:stride, kw:kw + outW * stride:stride])
    p = jnp.stack(cols, axis=2)
    p = p.reshape(B, C * k * k, outH * outW)
    return p, outH, outW


def _split3(t):
    bits = lax.bitcast_convert_type(t, jnp.uint32)
    hi = lax.bitcast_convert_type(bits & jnp.uint32(0xFFFF0000), jnp.float32)
    r1 = t - hi
    b1 = lax.bitcast_convert_type(r1, jnp.uint32)
    mid = lax.bitcast_convert_type(b1 & jnp.uint32(0xFFFF0000), jnp.float32)
    lo = r1 - mid
    return jnp.stack([hi.astype(jnp.bfloat16), mid.astype(jnp.bfloat16),
                      lo.astype(jnp.bfloat16)])


def _dot3(ohb, t3_ref):
    def d(piece):
        return lax.dot_general(ohb, piece, (((1,), (0,)), ((), ())),
                               preferred_element_type=jnp.float32)
    return (d(t3_ref[0]) + d(t3_ref[1])) + d(t3_ref[2])


def _prep_body(pdl_ref, c_ref, rc_ref, pt_ref, ct_ref, rct_ref):
    b = pl.program_id(0)
    pt_ref[...] = pdl_ref[...].T

    @pl.when(b == 0)
    def _():
        ct_ref[...] = c_ref[...].T
        rct_ref[...] = rc_ref[...].T


def _vq_body(p_ref, c3b_ref, rc3b_ref, c3_ref, rc3_ref, dc3_ref, drc3_ref,
             vals_ref, stats_ref, p3_ref, resid_ref, acc_ref):
    i = pl.program_id(0)
    iota = lax.broadcasted_iota(jnp.int32, (_TL, _K), 1).astype(jnp.float32)

    def dist_matrix(tab3_ref):
        def step(s, acc):
            for t in range(4):
                db = s * 4 + t
                p16 = p3_ref[db]
                c16 = tab3_ref[db]
                for j in range(16):
                    acc = acc + jnp.abs(p16[:, j:j + 1] - c16[j:j + 1, :])
            return acc
        return lax.fori_loop(0, _DB2 // 4, step,
                             jnp.zeros((_TL, _K), jnp.float32))

    def top2(acc):
        m1 = jnp.min(acc, axis=1)
        eq1 = acc == m1[:, None]
        i1 = jnp.min(jnp.where(eq1, iota, _BIGF), axis=1)
        accx = jnp.where(eq1, _BIGF, acc)
        m2 = jnp.min(accx, axis=1)
        i2 = jnp.min(jnp.where(accx == m2[:, None], iota, _BIGF), axis=1)
        return i1, i2

    def refine(q, t3_ref, ia, ib):
        ra = _dot3((iota == ia[:, None]).astype(jnp.bfloat16), t3_ref)
        rb = _dot3((iota == ib[:, None]).astype(jnp.bfloat16), t3_ref)
        da = jnp.sum(jnp.abs(q - ra), axis=1)
        db_ = jnp.sum(jnp.abs(q - rb), axis=1)
        swap = (db_ < da) | ((db_ == da) & (ib < ia))
        fi = jnp.where(swap, ib, ia)
        row = jnp.where(swap[:, None], rb, ra)
        return fi, row

    for db in range(_DB2):
        p3_ref[db] = p_ref[:, db * 16:(db + 1) * 16]

    acc1 = dist_matrix(c3b_ref)
    i1a, i1b = top2(acc1)
    p = p_ref[...]
    fi1, sel = refine(p, c3_ref, i1a, i1b)
    resid = p - sel
    resid_ref[...] = resid

    for db in range(_DB2):
        p3_ref[db] = resid[:, db * 16:(db + 1) * 16]

    acc2 = dist_matrix(rc3b_ref)
    i2a, i2b = top2(acc2)
    fi2, _ = refine(resid_ref[...], rc3_ref, i2a, i2b)

    oh1 = (iota == fi1[:, None]).astype(jnp.bfloat16)
    oh2 = (iota == fi2[:, None]).astype(jnp.bfloat16)
    vals = _dot3(oh1, dc3_ref) + _dot3(oh2, drc3_ref)
    vals_ref[...] = vals

    part = jnp.concatenate(
        [jnp.sum(vals, axis=0)[None, :],
         jnp.sum(vals * vals, axis=0)[None, :],
         jnp.zeros((6, _COUT), jnp.float32)], axis=0)

    @pl.when(i == 0)
    def _():
        acc_ref[...] = jnp.zeros_like(acc_ref)

    acc_ref[...] += part

    @pl.when(i == _NT - 1)
    def _():
        stats_ref[...] = acc_ref[...]


def _bn_body(vals_ref, stats_ref, g_ref, b_ref, out_ref):
    n = jnp.float32(vals_ref.shape[1] * pl.num_programs(0))
    s = stats_ref[0:1, :]
    ss = stats_ref[1:2, :]
    mean = s / n
    var = ss / n - mean * mean
    scale = g_ref[...] / jnp.sqrt(var + 1e-5)
    shift = b_ref[...] - mean * scale
    y = vals_ref[0] * scale + shift
    out_ref[0] = y.T


def kernel(x, centroids, residual_centroids, dot_centroids,
           dot_residual_centroids, bn_gamma, bn_beta):
    B = x.shape[0]
    patches_dl, outH, outW = _unfold_patches_dl(x)
    L = patches_dl.shape[2]

    patches, cT, rcT = pl.pallas_call(
        _prep_body,
        grid=(B,),
        in_specs=[
            pl.BlockSpec((_D, L), lambda b: (b, 0)),
            pl.BlockSpec((_K, _D), lambda b: (0, 0)),
            pl.BlockSpec((_K, _D), lambda b: (0, 0)),
        ],
        out_specs=[
            pl.BlockSpec((L, _D), lambda b: (b, 0)),
            pl.BlockSpec((_D, _K), lambda b: (0, 0)),
            pl.BlockSpec((_D, _K), lambda b: (0, 0)),
        ],
        out_shape=[
            jax.ShapeDtypeStruct((B * L, _D), jnp.float32),
            jax.ShapeDtypeStruct((_D, _K), jnp.float32),
            jax.ShapeDtypeStruct((_D, _K), jnp.float32),
        ],
    )(patches_dl.reshape(B * _D, L), centroids, residual_centroids)

    c3b = cT.reshape(_DB2, 16, _K)
    rc3b = rcT.reshape(_DB2, 16, _K)
    c3 = _split3(centroids)
    rc3 = _split3(residual_centroids)
    dc3 = _split3(dot_centroids)
    drc3 = _split3(dot_residual_centroids)

    full = lambda shape: pl.BlockSpec(shape, lambda i: tuple(0 for _ in shape))
    vals, stats = pl.pallas_call(
        _vq_body,
        grid=(_NT,),
        in_specs=[
            pl.BlockSpec((_TL, _D), lambda i: (i, 0)),
            full((_DB2, 16, _K)),
            full((_DB2, 16, _K)),
            full((3, _K, _D)),
            full((3, _K, _D)),
            full((3, _K, _COUT)),
            full((3, _K, _COUT)),
        ],
        out_specs=[
            pl.BlockSpec((_TL, _COUT), lambda i: (i, 0)),
            pl.BlockSpec((8, _COUT), lambda i: (0, 0)),
        ],
        out_shape=[
            jax.ShapeDtypeStruct((B * L, _COUT), jnp.float32),
            jax.ShapeDtypeStruct((8, _COUT), jnp.float32),
        ],
        scratch_shapes=[
            pltpu.VMEM((_DB2, _TL, 16), jnp.float32),
            pltpu.VMEM((_TL, _D), jnp.float32),
            pltpu.VMEM((8, _COUT), jnp.float32),
        ],
    )(patches, c3b, rc3b, c3, rc3, dc3, drc3)

    out = pl.pallas_call(
        _bn_body,
        grid=(B,),
        in_specs=[
            pl.BlockSpec((1, L, _COUT), lambda b: (b, 0, 0)),
            pl.BlockSpec((8, _COUT), lambda b: (0, 0)),
            pl.BlockSpec((1, _COUT), lambda b: (0, 0)),
            pl.BlockSpec((1, _COUT), lambda b: (0, 0)),
        ],
        out_specs=pl.BlockSpec((1, _COUT, L), lambda b: (b, 0, 0)),
        out_shape=jax.ShapeDtypeStruct((B, _COUT, L), jnp.float32),
    )(vals.reshape(B, L, _COUT), stats,
      bn_gamma.reshape(1, _COUT), bn_beta.reshape(1, _COUT))

    return out.reshape(B, _COUT, outH, outW)


# speedup vs baseline: 3.1067x; 1.0056x over previous
import jax
import jax.numpy as jnp
from jax import lax
from jax.experimental import pallas as pl
from jax.experimental.pallas import tpu as pltpu

_K = 512
_D = 576
_DB = _D // 8
_DB2 = _D // 16
_COUT = 128
_TL = 64
_NT = 49
_BIGF = 3e38


def _unfold_patches_dl(x):
    B, C, H, W = x.shape
    k, pad, stride = 3, 1, 1
    xp = jnp.pad(x, ((0, 0), (0, 0), (pad, pad), (pad, pad)))
    outH = (H + 2 * pad - k) // stride + 1
    outW = (W + 2 * pad - k) // stride + 1
    cols = []
    for kh in range(k):
        for kw in range(k):
            cols.append(xp[:, :, kh:kh + outH * stride:stride, kw:kw + outW * stride:stride])
    p = jnp.stack(cols, axis=2)
    p = p.reshape(B, C * k * k, outH * outW)
    return p, outH, outW


def _split3(t):
    bits = lax.bitcast_convert_type(t, jnp.uint32)
    hi = lax.bitcast_convert_type(bits & jnp.uint32(0xFFFF0000), jnp.float32)
    r1 = t - hi
    b1 = lax.bitcast_convert_type(r1, jnp.uint32)
    mid = lax.bitcast_convert_type(b1 & jnp.uint32(0xFFFF0000), jnp.float32)
    lo = r1 - mid
    return jnp.stack([hi.astype(jnp.bfloat16), mid.astype(jnp.bfloat16),
                      lo.astype(jnp.bfloat16)])


def _dot3(ohb, t3_ref):
    def d(piece):
        return lax.dot_general(ohb, piece, (((1,), (0,)), ((), ())),
                               preferred_element_type=jnp.float32)
    return (d(t3_ref[0]) + d(t3_ref[1])) + d(t3_ref[2])


def _prep_body(pdl_ref, c_ref, rc_ref, pt_ref, ct_ref, rct_ref):
    b = pl.program_id(0)
    pt_ref[...] = pdl_ref[...].T

    @pl.when(b == 0)
    def _():
        ct_ref[...] = c_ref[...].T
        rct_ref[...] = rc_ref[...].T


def _vq_body(p_ref, c3b_ref, rc3b_ref, c3_ref, rc3_ref, dc3_ref, drc3_ref,
             vals_ref, stats_ref, p3_ref, resid_ref, acc_ref):
    i = pl.program_id(0)
    iota = lax.broadcasted_iota(jnp.int32, (_TL, _K), 1).astype(jnp.float32)

    def dist_matrix(tab3_ref):
        def step(s, acc):
            for t in range(6):
                db = s * 6 + t
                p16 = p3_ref[db]
                c16 = tab3_ref[db]
                for j in range(16):
                    acc = acc + jnp.abs(p16[:, j:j + 1] - c16[j:j + 1, :])
            return acc
        return lax.fori_loop(0, _DB2 // 6, step,
                             jnp.zeros((_TL, _K), jnp.float32))

    def top2(acc):
        m1 = jnp.min(acc, axis=1)
        eq1 = acc == m1[:, None]
        i1 = jnp.min(jnp.where(eq1, iota, _BIGF), axis=1)
        accx = jnp.where(eq1, _BIGF, acc)
        m2 = jnp.min(accx, axis=1)
        i2 = jnp.min(jnp.where(accx == m2[:, None], iota, _BIGF), axis=1)
        return i1, i2

    def refine(q, t3_ref, ia, ib):
        ra = _dot3((iota == ia[:, None]).astype(jnp.bfloat16), t3_ref)
        rb = _dot3((iota == ib[:, None]).astype(jnp.bfloat16), t3_ref)
        da = jnp.sum(jnp.abs(q - ra), axis=1)
        db_ = jnp.sum(jnp.abs(q - rb), axis=1)
        swap = (db_ < da) | ((db_ == da) & (ib < ia))
        fi = jnp.where(swap, ib, ia)
        row = jnp.where(swap[:, None], rb, ra)
        return fi, row

    for db in range(_DB2):
        p3_ref[db] = p_ref[:, db * 16:(db + 1) * 16]

    acc1 = dist_matrix(c3b_ref)
    i1a, i1b = top2(acc1)
    p = p_ref[...]
    fi1, sel = refine(p, c3_ref, i1a, i1b)
    resid = p - sel
    resid_ref[...] = resid

    for db in range(_DB2):
        p3_ref[db] = resid[:, db * 16:(db + 1) * 16]

    acc2 = dist_matrix(rc3b_ref)
    i2a, i2b = top2(acc2)
    fi2, _ = refine(resid_ref[...], rc3_ref, i2a, i2b)

    oh1 = (iota == fi1[:, None]).astype(jnp.bfloat16)
    oh2 = (iota == fi2[:, None]).astype(jnp.bfloat16)
    vals = _dot3(oh1, dc3_ref) + _dot3(oh2, drc3_ref)
    vals_ref[...] = vals

    part = jnp.concatenate(
        [jnp.sum(vals, axis=0)[None, :],
         jnp.sum(vals * vals, axis=0)[None, :],
         jnp.zeros((6, _COUT), jnp.float32)], axis=0)

    @pl.when(i == 0)
    def _():
        acc_ref[...] = jnp.zeros_like(acc_ref)

    acc_ref[...] += part

    @pl.when(i == _NT - 1)
    def _():
        stats_ref[...] = acc_ref[...]


def _bn_body(vals_ref, stats_ref, g_ref, b_ref, out_ref):
    n = jnp.float32(vals_ref.shape[1] * pl.num_programs(0))
    s = stats_ref[0:1, :]
    ss = stats_ref[1:2, :]
    mean = s / n
    var = ss / n - mean * mean
    scale = g_ref[...] / jnp.sqrt(var + 1e-5)
    shift = b_ref[...] - mean * scale
    y = vals_ref[0] * scale + shift
    out_ref[0] = y.T


def kernel(x, centroids, residual_centroids, dot_centroids,
           dot_residual_centroids, bn_gamma, bn_beta):
    B = x.shape[0]
    patches_dl, outH, outW = _unfold_patches_dl(x)
    L = patches_dl.shape[2]

    patches, cT, rcT = pl.pallas_call(
        _prep_body,
        grid=(B,),
        in_specs=[
            pl.BlockSpec((_D, L), lambda b: (b, 0)),
            pl.BlockSpec((_K, _D), lambda b: (0, 0)),
            pl.BlockSpec((_K, _D), lambda b: (0, 0)),
        ],
        out_specs=[
            pl.BlockSpec((L, _D), lambda b: (b, 0)),
            pl.BlockSpec((_D, _K), lambda b: (0, 0)),
            pl.BlockSpec((_D, _K), lambda b: (0, 0)),
        ],
        out_shape=[
            jax.ShapeDtypeStruct((B * L, _D), jnp.float32),
            jax.ShapeDtypeStruct((_D, _K), jnp.float32),
            jax.ShapeDtypeStruct((_D, _K), jnp.float32),
        ],
    )(patches_dl.reshape(B * _D, L), centroids, residual_centroids)

    c3b = cT.reshape(_DB2, 16, _K)
    rc3b = rcT.reshape(_DB2, 16, _K)
    c3 = _split3(centroids)
    rc3 = _split3(residual_centroids)
    dc3 = _split3(dot_centroids)
    drc3 = _split3(dot_residual_centroids)

    full = lambda shape: pl.BlockSpec(shape, lambda i: tuple(0 for _ in shape))
    vals, stats = pl.pallas_call(
        _vq_body,
        grid=(_NT,),
        in_specs=[
            pl.BlockSpec((_TL, _D), lambda i: (i, 0)),
            full((_DB2, 16, _K)),
            full((_DB2, 16, _K)),
            full((3, _K, _D)),
            full((3, _K, _D)),
            full((3, _K, _COUT)),
            full((3, _K, _COUT)),
        ],
        out_specs=[
            pl.BlockSpec((_TL, _COUT), lambda i: (i, 0)),
            pl.BlockSpec((8, _COUT), lambda i: (0, 0)),
        ],
        out_shape=[
            jax.ShapeDtypeStruct((B * L, _COUT), jnp.float32),
            jax.ShapeDtypeStruct((8, _COUT), jnp.float32),
        ],
        scratch_shapes=[
            pltpu.VMEM((_DB2, _TL, 16), jnp.float32),
            pltpu.VMEM((_TL, _D), jnp.float32),
            pltpu.VMEM((8, _COUT), jnp.float32),
        ],
    )(patches, c3b, rc3b, c3, rc3, dc3, drc3)

    out = pl.pallas_call(
        _bn_body,
        grid=(B,),
        in_specs=[
            pl.BlockSpec((1, L, _COUT), lambda b: (b, 0, 0)),
            pl.BlockSpec((8, _COUT), lambda b: (0, 0)),
            pl.BlockSpec((1, _COUT), lambda b: (0, 0)),
            pl.BlockSpec((1, _COUT), lambda b: (0, 0)),
        ],
        out_specs=pl.BlockSpec((1, _COUT, L), lambda b: (b, 0, 0)),
        out_shape=jax.ShapeDtypeStruct((B, _COUT, L), jnp.float32),
    )(vals.reshape(B, L, _COUT), stats,
      bn_gamma.reshape(1, _COUT), bn_beta.reshape(1, _COUT))

    return out.reshape(B, _COUT, outH, outW)


# speedup vs baseline: 3.1277x; 1.0068x over previous
import jax
import jax.numpy as jnp
from jax import lax
from jax.experimental import pallas as pl
from jax.experimental.pallas import tpu as pltpu

_K = 512
_D = 576
_DB = _D // 8
_DB2 = _D // 16
_COUT = 128
_TL = 64
_NT = 49
_BIGF = 3e38


def _unfold_patches_dl(x):
    B, C, H, W = x.shape
    k, pad, stride = 3, 1, 1
    xp = jnp.pad(x, ((0, 0), (0, 0), (pad, pad), (pad, pad)))
    outH = (H + 2 * pad - k) // stride + 1
    outW = (W + 2 * pad - k) // stride + 1
    cols = []
    for kh in range(k):
        for kw in range(k):
            cols.append(xp[:, :, kh:kh + outH * stride:stride, kw:kw + outW * stride:stride])
    p = jnp.stack(cols, axis=2)
    p = p.reshape(B, C * k * k, outH * outW)
    return p, outH, outW


def _split3(t):
    bits = lax.bitcast_convert_type(t, jnp.uint32)
    hi = lax.bitcast_convert_type(bits & jnp.uint32(0xFFFF0000), jnp.float32)
    r1 = t - hi
    b1 = lax.bitcast_convert_type(r1, jnp.uint32)
    mid = lax.bitcast_convert_type(b1 & jnp.uint32(0xFFFF0000), jnp.float32)
    lo = r1 - mid
    return jnp.stack([hi.astype(jnp.bfloat16), mid.astype(jnp.bfloat16),
                      lo.astype(jnp.bfloat16)])


def _dot3(ohb, t3_ref):
    def d(piece):
        return lax.dot_general(ohb, piece, (((1,), (0,)), ((), ())),
                               preferred_element_type=jnp.float32)
    return (d(t3_ref[0]) + d(t3_ref[1])) + d(t3_ref[2])


def _prep_body(pdl_ref, c_ref, rc_ref, pt_ref, ct_ref, rct_ref):
    b = pl.program_id(0)
    pt_ref[...] = pdl_ref[...].T

    @pl.when(b == 0)
    def _():
        ct_ref[...] = c_ref[...].T
        rct_ref[...] = rc_ref[...].T


def _vq_body(p_ref, c3b_ref, rc3b_ref, c3_ref, rc3_ref, dc3_ref, drc3_ref,
             vals_ref, stats_ref, p3_ref, resid_ref, acc_ref):
    i = pl.program_id(0)
    iota = lax.broadcasted_iota(jnp.int32, (_TL, _K), 1).astype(jnp.float32)

    def dist_matrix(tab3_ref):
        acc = jnp.zeros((_TL, _K), jnp.float32)
        for db in range(_DB2):
            p16 = p3_ref[db]
            c16 = tab3_ref[db]
            for j in range(16):
                acc = acc + jnp.abs(p16[:, j:j + 1] - c16[j:j + 1, :])
        return acc

    def top2(acc):
        m1 = jnp.min(acc, axis=1)
        eq1 = acc == m1[:, None]
        i1 = jnp.min(jnp.where(eq1, iota, _BIGF), axis=1)
        accx = jnp.where(eq1, _BIGF, acc)
        m2 = jnp.min(accx, axis=1)
        i2 = jnp.min(jnp.where(accx == m2[:, None], iota, _BIGF), axis=1)
        return i1, i2

    def refine(q, t3_ref, ia, ib):
        ra = _dot3((iota == ia[:, None]).astype(jnp.bfloat16), t3_ref)
        rb = _dot3((iota == ib[:, None]).astype(jnp.bfloat16), t3_ref)
        da = jnp.sum(jnp.abs(q - ra), axis=1)
        db_ = jnp.sum(jnp.abs(q - rb), axis=1)
        swap = (db_ < da) | ((db_ == da) & (ib < ia))
        fi = jnp.where(swap, ib, ia)
        row = jnp.where(swap[:, None], rb, ra)
        return fi, row

    for db in range(_DB2):
        p3_ref[db] = p_ref[:, db * 16:(db + 1) * 16]

    acc1 = dist_matrix(c3b_ref)
    i1a, i1b = top2(acc1)
    p = p_ref[...]
    fi1, sel = refine(p, c3_ref, i1a, i1b)
    resid = p - sel
    resid_ref[...] = resid

    for db in range(_DB2):
        p3_ref[db] = resid[:, db * 16:(db + 1) * 16]

    acc2 = dist_matrix(rc3b_ref)
    i2a, i2b = top2(acc2)
    fi2, _ = refine(resid_ref[...], rc3_ref, i2a, i2b)

    oh1 = (iota == fi1[:, None]).astype(jnp.bfloat16)
    oh2 = (iota == fi2[:, None]).astype(jnp.bfloat16)
    vals = _dot3(oh1, dc3_ref) + _dot3(oh2, drc3_ref)
    vals_ref[...] = vals

    part = jnp.concatenate(
        [jnp.sum(vals, axis=0)[None, :],
         jnp.sum(vals * vals, axis=0)[None, :],
         jnp.zeros((6, _COUT), jnp.float32)], axis=0)

    @pl.when(i == 0)
    def _():
        acc_ref[...] = jnp.zeros_like(acc_ref)

    acc_ref[...] += part

    @pl.when(i == _NT - 1)
    def _():
        stats_ref[...] = acc_ref[...]


def _bn_body(vals_ref, stats_ref, g_ref, b_ref, out_ref):
    n = jnp.float32(vals_ref.shape[1] * pl.num_programs(0))
    s = stats_ref[0:1, :]
    ss = stats_ref[1:2, :]
    mean = s / n
    var = ss / n - mean * mean
    scale = g_ref[...] / jnp.sqrt(var + 1e-5)
    shift = b_ref[...] - mean * scale
    y = vals_ref[0] * scale + shift
    out_ref[0] = y.T


def kernel(x, centroids, residual_centroids, dot_centroids,
           dot_residual_centroids, bn_gamma, bn_beta):
    B = x.shape[0]
    patches_dl, outH, outW = _unfold_patches_dl(x)
    L = patches_dl.shape[2]

    patches, cT, rcT = pl.pallas_call(
        _prep_body,
        grid=(B,),
        in_specs=[
            pl.BlockSpec((_D, L), lambda b: (b, 0)),
            pl.BlockSpec((_K, _D), lambda b: (0, 0)),
            pl.BlockSpec((_K, _D), lambda b: (0, 0)),
        ],
        out_specs=[
            pl.BlockSpec((L, _D), lambda b: (b, 0)),
            pl.BlockSpec((_D, _K), lambda b: (0, 0)),
            pl.BlockSpec((_D, _K), lambda b: (0, 0)),
        ],
        out_shape=[
            jax.ShapeDtypeStruct((B * L, _D), jnp.float32),
            jax.ShapeDtypeStruct((_D, _K), jnp.float32),
            jax.ShapeDtypeStruct((_D, _K), jnp.float32),
        ],
    )(patches_dl.reshape(B * _D, L), centroids, residual_centroids)

    c3b = cT.reshape(_DB2, 16, _K)
    rc3b = rcT.reshape(_DB2, 16, _K)
    c3 = _split3(centroids)
    rc3 = _split3(residual_centroids)
    dc3 = _split3(dot_centroids)
    drc3 = _split3(dot_residual_centroids)

    full = lambda shape: pl.BlockSpec(shape, lambda i: tuple(0 for _ in shape))
    vals, stats = pl.pallas_call(
        _vq_body,
        grid=(_NT,),
        in_specs=[
            pl.BlockSpec((_TL, _D), lambda i: (i, 0)),
            full((_DB2, 16, _K)),
            full((_DB2, 16, _K)),
            full((3, _K, _D)),
            full((3, _K, _D)),
            full((3, _K, _COUT)),
            full((3, _K, _COUT)),
        ],
        out_specs=[
            pl.BlockSpec((_TL, _COUT), lambda i: (i, 0)),
            pl.BlockSpec((8, _COUT), lambda i: (0, 0)),
        ],
        out_shape=[
            jax.ShapeDtypeStruct((B * L, _COUT), jnp.float32),
            jax.ShapeDtypeStruct((8, _COUT), jnp.float32),
        ],
        scratch_shapes=[
            pltpu.VMEM((_DB2, _TL, 16), jnp.float32),
            pltpu.VMEM((_TL, _D), jnp.float32),
            pltpu.VMEM((8, _COUT), jnp.float32),
        ],
    )(patches, c3b, rc3b, c3, rc3, dc3, drc3)

    out = pl.pallas_call(
        _bn_body,
        grid=(B,),
        in_specs=[
            pl.BlockSpec((1, L, _COUT), lambda b: (b, 0, 0)),
            pl.BlockSpec((8, _COUT), lambda b: (0, 0)),
            pl.BlockSpec((1, _COUT), lambda b: (0, 0)),
            pl.BlockSpec((1, _COUT), lambda b: (0, 0)),
        ],
        out_specs=pl.BlockSpec((1, _COUT, L), lambda b: (b, 0, 0)),
        out_shape=jax.ShapeDtypeStruct((B, _COUT, L), jnp.float32),
    )(vals.reshape(B, L, _COUT), stats,
      bn_gamma.reshape(1, _COUT), bn_beta.reshape(1, _COUT))

    return out.reshape(B, _COUT, outH, outW)
